# baseline (speedup 1.0000x reference)
import jax
import jax.numpy as jnp
from jax.experimental import pallas as pl
from jax.experimental.pallas import tpu as pltpu

EMB = 24
ROWS = 8


def _lookup_kernel(idx_ref, rows_ref, uv_ref, out_ref):
    sub = idx_ref[0] % ROWS
    rows = rows_ref[...]
    uv = uv_ref[...]
    mask = jax.lax.broadcasted_iota(jnp.int32, (ROWS, EMB), 0) == sub
    picked = jnp.where(mask, rows, 0.0)
    s = jnp.sum(picked * uv, keepdims=True).reshape(1, 1)
    out_ref[...] = jax.nn.sigmoid(s)


def kernel(input, matrix, user_vector):
    idx = jnp.asarray(input, jnp.int32).reshape((1,))
    out = pl.pallas_call(
        _lookup_kernel,
        grid_spec=pltpu.PrefetchScalarGridSpec(
            num_scalar_prefetch=1,
            grid=(1,),
            in_specs=[
                pl.BlockSpec((ROWS, EMB), lambda i, idx_ref: (idx_ref[0] // ROWS, 0)),
                pl.BlockSpec((1, EMB), lambda i, idx_ref: (0, 0)),
            ],
            out_specs=pl.BlockSpec((1, 1), lambda i, idx_ref: (0, 0)),
        ),
        out_shape=jax.ShapeDtypeStruct((1, 1), jnp.float32),
    )(idx, matrix, user_vector)
    return out.reshape((1,))
